# initial kernel scaffold (unmeasured)
import jax
import jax.numpy as jnp
from jax import lax
from jax.experimental import pallas as pl
from jax.experimental.pallas import tpu as pltpu

N_DEV = 8
M = 1024
D = 1024
F = 4096
FC = 1024
NFC = F // FC


def kernel(x, W1, W2):
    def body(x_ref, w1_hbm, w2_hbm, out_ref,
             xbuf, accbuf, asend, w1buf, w2buf,
             x_send_sems, x_recv_sems, a_send_sems, a_recv_sems, ldma_sems):
        my = lax.axis_index("i")
        left = lax.rem(my + N_DEV - 1, N_DEV)
        right = lax.rem(my + 1, N_DEV)

        barrier = pltpu.get_barrier_semaphore()
        pl.semaphore_signal(barrier, inc=1, device_id=(left,),
                            device_id_type=pl.DeviceIdType.MESH)
        pl.semaphore_signal(barrier, inc=1, device_id=(right,),
                            device_id_type=pl.DeviceIdType.MESH)
        pl.semaphore_wait(barrier, 2)

        def compute_partial(x_val):
            def fc_body(fc, p):
                c1 = pltpu.make_async_copy(
                    w1_hbm.at[:, pl.ds(fc * FC, FC)], w1buf, ldma_sems.at[0])
                c2 = pltpu.make_async_copy(
                    w2_hbm.at[pl.ds(fc * FC, FC), :], w2buf, ldma_sems.at[1])
                c1.start()
                c2.start()
                c1.wait()
                c2.wait()
                h = jnp.dot(x_val, w1buf[...],
                            preferred_element_type=jnp.float32)
                h = h * jax.nn.sigmoid(h)
                return p + jnp.dot(h, w2buf[...],
                                   preferred_element_type=jnp.float32)
            return lax.fori_loop(0, NFC, fc_body,
                                 jnp.zeros((M, D), jnp.float32))

        for k in range(N_DEV - 1):
            src = x_ref if k == 0 else xbuf.at[(k - 1) % 2]
            rdma_x = pltpu.make_async_remote_copy(
                src_ref=src,
                dst_ref=xbuf.at[k % 2],
                send_sem=x_send_sems.at[k % 2],
                recv_sem=x_recv_sems.at[k % 2],
                device_id=(right,),
                device_id_type=pl.DeviceIdType.MESH,
            )
            rdma_x.start()
            rdma_x.wait()

            p = compute_partial(xbuf[k % 2])

            if k == 0:
                asend[...] = p
            else:
                asend[...] = accbuf[(k - 1) % 2] + p
            rdma_a = pltpu.make_async_remote_copy(
                src_ref=asend,
                dst_ref=accbuf.at[k % 2],
                send_sem=a_send_sems.at[k % 2],
                recv_sem=a_recv_sems.at[k % 2],
                device_id=(right,),
                device_id_type=pl.DeviceIdType.MESH,
            )
            rdma_a.start()
            rdma_a.wait()

        out_ref[...] = accbuf[(N_DEV - 2) % 2] + compute_partial(x_ref[...])

    return pl.pallas_call(
        body,
        out_shape=jax.ShapeDtypeStruct((M, D), jnp.float32),
        in_specs=[
            pl.BlockSpec(memory_space=pltpu.MemorySpace.VMEM),
            pl.BlockSpec(memory_space=pltpu.MemorySpace.ANY),
            pl.BlockSpec(memory_space=pltpu.MemorySpace.ANY),
        ],
        out_specs=pl.BlockSpec(memory_space=pltpu.MemorySpace.VMEM),
        scratch_shapes=[
            pltpu.VMEM((2, M, D), jnp.float32),
            pltpu.VMEM((2, M, D), jnp.float32),
            pltpu.VMEM((M, D), jnp.float32),
            pltpu.VMEM((D, FC), jnp.float32),
            pltpu.VMEM((FC, D), jnp.float32),
            pltpu.SemaphoreType.DMA((2,)),
            pltpu.SemaphoreType.DMA((2,)),
            pltpu.SemaphoreType.DMA((2,)),
            pltpu.SemaphoreType.DMA((2,)),
            pltpu.SemaphoreType.DMA((2,)),
        ],
        compiler_params=pltpu.CompilerParams(collective_id=0),
    )(x, W1, W2)


# baseline (device time: 957742 ns/iter reference)
import jax
import jax.numpy as jnp
from jax import lax
from jax.experimental import pallas as pl
from jax.experimental.pallas import tpu as pltpu

N_DEV = 8
M = 1024
D = 1024
F = 4096
FC = 1024
NFC = F // FC


def kernel(x, W1, W2):
    def body(x_ref, w1_hbm, w2_hbm, out_ref,
             xbuf, accbuf, asend, w1buf, w2buf,
             x_send_sems, x_recv_sems, a_send_sems, a_recv_sems, ldma_sems):
        my = lax.axis_index("i")
        left = lax.rem(my + N_DEV - 1, N_DEV)
        right = lax.rem(my + 1, N_DEV)

        barrier = pltpu.get_barrier_semaphore()
        pl.semaphore_signal(barrier, inc=1, device_id=(left,),
                            device_id_type=pl.DeviceIdType.MESH)
        pl.semaphore_signal(barrier, inc=1, device_id=(right,),
                            device_id_type=pl.DeviceIdType.MESH)
        pl.semaphore_wait(barrier, 2)

        def compute_partial(x_val):
            def fc_body(fc, p):
                c1 = pltpu.make_async_copy(
                    w1_hbm.at[:, pl.ds(fc * FC, FC)], w1buf, ldma_sems.at[0])
                c2 = pltpu.make_async_copy(
                    w2_hbm.at[pl.ds(fc * FC, FC), :], w2buf, ldma_sems.at[1])
                c1.start()
                c2.start()
                c1.wait()
                c2.wait()
                h = jnp.dot(x_val, w1buf[...],
                            preferred_element_type=jnp.float32)
                h = h * jax.nn.sigmoid(h)
                return p + jnp.dot(h, w2buf[...],
                                   preferred_element_type=jnp.float32)
            return lax.fori_loop(0, NFC, fc_body,
                                 jnp.zeros((M, D), jnp.float32))

        for k in range(N_DEV - 1):
            src = x_ref if k == 0 else xbuf.at[(k - 1) % 2]
            rdma_x = pltpu.make_async_remote_copy(
                src_ref=src,
                dst_ref=xbuf.at[k % 2],
                send_sem=x_send_sems.at[k % 2],
                recv_sem=x_recv_sems.at[k % 2],
                device_id=(right,),
                device_id_type=pl.DeviceIdType.MESH,
            )
            rdma_x.start()
            rdma_x.wait()

            p = compute_partial(xbuf[k % 2])

            if k == 0:
                asend[...] = p
            else:
                asend[...] = accbuf[(k - 1) % 2] + p
            rdma_a = pltpu.make_async_remote_copy(
                src_ref=asend,
                dst_ref=accbuf.at[k % 2],
                send_sem=a_send_sems.at[k % 2],
                recv_sem=a_recv_sems.at[k % 2],
                device_id=(right,),
                device_id_type=pl.DeviceIdType.MESH,
            )
            rdma_a.start()
            rdma_a.wait()

        out_ref[...] = accbuf[(N_DEV - 2) % 2] + compute_partial(x_ref[...])

    return pl.pallas_call(
        body,
        out_shape=jax.ShapeDtypeStruct((M, D), jnp.float32),
        in_specs=[
            pl.BlockSpec(memory_space=pltpu.MemorySpace.VMEM),
            pl.BlockSpec(memory_space=pl.ANY),
            pl.BlockSpec(memory_space=pl.ANY),
        ],
        out_specs=pl.BlockSpec(memory_space=pltpu.MemorySpace.VMEM),
        scratch_shapes=[
            pltpu.VMEM((2, M, D), jnp.float32),
            pltpu.VMEM((2, M, D), jnp.float32),
            pltpu.VMEM((M, D), jnp.float32),
            pltpu.VMEM((D, FC), jnp.float32),
            pltpu.VMEM((FC, D), jnp.float32),
            pltpu.SemaphoreType.DMA((2,)),
            pltpu.SemaphoreType.DMA((2,)),
            pltpu.SemaphoreType.DMA((2,)),
            pltpu.SemaphoreType.DMA((2,)),
            pltpu.SemaphoreType.DMA((2,)),
        ],
        compiler_params=pltpu.CompilerParams(
            collective_id=0,
            vmem_limit_bytes=56 * 1024 * 1024,
        ),
    )(x, W1, W2)


# device time: 372959 ns/iter; 2.5680x vs baseline; 2.5680x over previous
import jax
import jax.numpy as jnp
from jax import lax
from jax.experimental import pallas as pl
from jax.experimental.pallas import tpu as pltpu

N_DEV = 8
NH = N_DEV - 1
M = 1024
MH = 512
D = 1024
F = 4096
FC = 1024
NFC = F // FC

_MESH = pl.DeviceIdType.MESH


def kernel(x, W1, W2):
    def body(x_ref, w1_hbm, w2_hbm, out_ref,
             xbR, xbL, abR, abL, asR, asL, w1buf, w2buf,
             xsR, xrR, xsL, xrL, asRs, arR, asLs, arL, ldma,
             xcredR, xcredL, acredR, acredL):
        my = lax.axis_index("i")
        left = lax.rem(my + N_DEV - 1, N_DEV)
        right = lax.rem(my + 1, N_DEV)

        barrier = pltpu.get_barrier_semaphore()
        pl.semaphore_signal(barrier, inc=1, device_id=(left,),
                            device_id_type=_MESH)
        pl.semaphore_signal(barrier, inc=1, device_id=(right,),
                            device_id_type=_MESH)
        pl.semaphore_wait(barrier, 2)

        def xR_hop(k):
            src = x_ref.at[pl.ds(0, MH), :] if k == 0 else xbR.at[(k - 1) % 2]
            return pltpu.make_async_remote_copy(
                src_ref=src, dst_ref=xbR.at[k % 2],
                send_sem=xsR.at[k % 2], recv_sem=xrR.at[k % 2],
                device_id=(right,), device_id_type=_MESH)

        def xL_hop(k):
            src = x_ref.at[pl.ds(MH, MH), :] if k == 0 else xbL.at[(k - 1) % 2]
            return pltpu.make_async_remote_copy(
                src_ref=src, dst_ref=xbL.at[k % 2],
                send_sem=xsL.at[k % 2], recv_sem=xrL.at[k % 2],
                device_id=(left,), device_id_type=_MESH)

        def aR_hop(k):
            return pltpu.make_async_remote_copy(
                src_ref=asR, dst_ref=abR.at[k % 2],
                send_sem=asRs.at[k % 2], recv_sem=arR.at[k % 2],
                device_id=(right,), device_id_type=_MESH)

        def aL_hop(k):
            return pltpu.make_async_remote_copy(
                src_ref=asL, dst_ref=abL.at[k % 2],
                send_sem=asLs.at[k % 2], recv_sem=arL.at[k % 2],
                device_id=(left,), device_id_type=_MESH)

        def partials(xr_val, xl_val):
            def fc_body(fc, carry):
                pr, plf = carry
                c1 = pltpu.make_async_copy(
                    w1_hbm.at[:, pl.ds(fc * FC, FC)], w1buf, ldma.at[0])
                c2 = pltpu.make_async_copy(
                    w2_hbm.at[pl.ds(fc * FC, FC), :], w2buf, ldma.at[1])
                c1.start()
                c2.start()
                c1.wait()
                c2.wait()
                w1c = w1buf[...]
                w2c = w2buf[...]
                hr = jnp.dot(xr_val, w1c, preferred_element_type=jnp.float32)
                hr = hr * jax.nn.sigmoid(hr)
                hl = jnp.dot(xl_val, w1c, preferred_element_type=jnp.float32)
                hl = hl * jax.nn.sigmoid(hl)
                pr = pr + jnp.dot(hr, w2c, preferred_element_type=jnp.float32)
                plf = plf + jnp.dot(hl, w2c, preferred_element_type=jnp.float32)
                return pr, plf
            z = jnp.zeros((MH, D), jnp.float32)
            return lax.fori_loop(0, NFC, fc_body, (z, z))

        xR_hop(0).start()
        xL_hop(0).start()

        for k in range(NH):
            xR_hop(k).wait_recv()
            xL_hop(k).wait_recv()

            if k + 1 < NH:
                if k + 1 >= 2:
                    pl.semaphore_wait(xcredR, 1)
                    pl.semaphore_wait(xcredL, 1)
                xR_hop(k + 1).start()
                xL_hop(k + 1).start()

            pr, plf = partials(xbR[k % 2], xbL[k % 2])

            if k >= 1:
                aR_hop(k - 1).wait_recv()
                aL_hop(k - 1).wait_recv()
                aR_hop(k - 1).wait_send()
                aL_hop(k - 1).wait_send()
                asR[...] = abR[(k - 1) % 2] + pr
                asL[...] = abL[(k - 1) % 2] + plf
                if k <= 5:
                    pl.semaphore_signal(acredR, inc=1, device_id=(left,),
                                        device_id_type=_MESH)
                    pl.semaphore_signal(acredL, inc=1, device_id=(right,),
                                        device_id_type=_MESH)
            else:
                asR[...] = pr
                asL[...] = plf

            if k >= 2:
                pl.semaphore_wait(acredR, 1)
                pl.semaphore_wait(acredL, 1)
            aR_hop(k).start()
            aL_hop(k).start()

            if k == 0:
                xR_hop(0).wait_send()
                xL_hop(0).wait_send()
            if k + 1 < NH:
                xR_hop(k + 1).wait_send()
                xL_hop(k + 1).wait_send()
                if k <= 4:
                    pl.semaphore_signal(xcredR, inc=1, device_id=(left,),
                                        device_id_type=_MESH)
                    pl.semaphore_signal(xcredL, inc=1, device_id=(right,),
                                        device_id_type=_MESH)

        xv = x_ref[...]
        pr_own, pl_own = partials(xv[:MH], xv[MH:])

        aR_hop(NH - 1).wait_recv()
        aL_hop(NH - 1).wait_recv()
        out_ref[pl.ds(0, MH), :] = abR[(NH - 1) % 2] + pr_own
        out_ref[pl.ds(MH, MH), :] = abL[(NH - 1) % 2] + pl_own
        aR_hop(NH - 1).wait_send()
        aL_hop(NH - 1).wait_send()

    return pl.pallas_call(
        body,
        out_shape=jax.ShapeDtypeStruct((M, D), jnp.float32),
        in_specs=[
            pl.BlockSpec(memory_space=pltpu.MemorySpace.VMEM),
            pl.BlockSpec(memory_space=pl.ANY),
            pl.BlockSpec(memory_space=pl.ANY),
        ],
        out_specs=pl.BlockSpec(memory_space=pltpu.MemorySpace.VMEM),
        scratch_shapes=[
            pltpu.VMEM((2, MH, D), jnp.float32),
            pltpu.VMEM((2, MH, D), jnp.float32),
            pltpu.VMEM((2, MH, D), jnp.float32),
            pltpu.VMEM((2, MH, D), jnp.float32),
            pltpu.VMEM((MH, D), jnp.float32),
            pltpu.VMEM((MH, D), jnp.float32),
            pltpu.VMEM((D, FC), jnp.float32),
            pltpu.VMEM((FC, D), jnp.float32),
            pltpu.SemaphoreType.DMA((2,)),
            pltpu.SemaphoreType.DMA((2,)),
            pltpu.SemaphoreType.DMA((2,)),
            pltpu.SemaphoreType.DMA((2,)),
            pltpu.SemaphoreType.DMA((2,)),
            pltpu.SemaphoreType.DMA((2,)),
            pltpu.SemaphoreType.DMA((2,)),
            pltpu.SemaphoreType.DMA((2,)),
            pltpu.SemaphoreType.DMA((2,)),
            pltpu.SemaphoreType.REGULAR,
            pltpu.SemaphoreType.REGULAR,
            pltpu.SemaphoreType.REGULAR,
            pltpu.SemaphoreType.REGULAR,
        ],
        compiler_params=pltpu.CompilerParams(
            collective_id=0,
            vmem_limit_bytes=56 * 1024 * 1024,
        ),
    )(x, W1, W2)


# device time: 345087 ns/iter; 2.7754x vs baseline; 1.0808x over previous
import jax
import jax.numpy as jnp
from jax import lax
from jax.experimental import pallas as pl
from jax.experimental.pallas import tpu as pltpu

N_DEV = 8
NH = N_DEV - 1
M = 1024
MH = 512
D = 1024
F = 4096
FC = 512
NFC = F // FC

_MESH = pl.DeviceIdType.MESH


def kernel(x, W1, W2):
    def body(x_ref, w1_hbm, w2_hbm, out_ref,
             xbR, xbL, abR, abL, asR, asL, w1buf, w2buf,
             xsR, xrR, xsL, xrL, asRs, arR, asLs, arL, ldma,
             xcredR, xcredL, acredR, acredL):
        my = lax.axis_index("i")
        left = lax.rem(my + N_DEV - 1, N_DEV)
        right = lax.rem(my + 1, N_DEV)

        barrier = pltpu.get_barrier_semaphore()
        pl.semaphore_signal(barrier, inc=1, device_id=(left,),
                            device_id_type=_MESH)
        pl.semaphore_signal(barrier, inc=1, device_id=(right,),
                            device_id_type=_MESH)
        pl.semaphore_wait(barrier, 2)

        def xR_hop(k):
            src = x_ref.at[pl.ds(0, MH), :] if k == 0 else xbR.at[(k - 1) % 2]
            return pltpu.make_async_remote_copy(
                src_ref=src, dst_ref=xbR.at[k % 2],
                send_sem=xsR.at[k % 2], recv_sem=xrR.at[k % 2],
                device_id=(right,), device_id_type=_MESH)

        def xL_hop(k):
            src = x_ref.at[pl.ds(MH, MH), :] if k == 0 else xbL.at[(k - 1) % 2]
            return pltpu.make_async_remote_copy(
                src_ref=src, dst_ref=xbL.at[k % 2],
                send_sem=xsL.at[k % 2], recv_sem=xrL.at[k % 2],
                device_id=(left,), device_id_type=_MESH)

        def aR_hop(k):
            return pltpu.make_async_remote_copy(
                src_ref=asR, dst_ref=abR.at[k % 2],
                send_sem=asRs.at[k % 2], recv_sem=arR.at[k % 2],
                device_id=(right,), device_id_type=_MESH)

        def aL_hop(k):
            return pltpu.make_async_remote_copy(
                src_ref=asL, dst_ref=abL.at[k % 2],
                send_sem=asLs.at[k % 2], recv_sem=arL.at[k % 2],
                device_id=(left,), device_id_type=_MESH)

        def wchunk_dma(fc, slot):
            c1 = pltpu.make_async_copy(
                w1_hbm.at[:, pl.ds(fc * FC, FC)], w1buf.at[slot],
                ldma.at[0, slot])
            c2 = pltpu.make_async_copy(
                w2_hbm.at[pl.ds(fc * FC, FC), :], w2buf.at[slot],
                ldma.at[1, slot])
            return c1, c2

        def partials(xr_val, xl_val):
            c1, c2 = wchunk_dma(0, 0)
            c1.start()
            c2.start()

            def fc_body(fc, carry):
                pr, plf = carry
                slot = lax.rem(fc, 2)
                c1, c2 = wchunk_dma(fc, slot)
                c1.wait()
                c2.wait()

                @pl.when(fc + 1 < NFC)
                def _():
                    n1, n2 = wchunk_dma(fc + 1, lax.rem(fc + 1, 2))
                    n1.start()
                    n2.start()

                w1c = w1buf[slot]
                w2c = w2buf[slot]
                hr = jnp.dot(xr_val, w1c, preferred_element_type=jnp.float32)
                hr = hr * jax.nn.sigmoid(hr)
                hl = jnp.dot(xl_val, w1c, preferred_element_type=jnp.float32)
                hl = hl * jax.nn.sigmoid(hl)
                pr = pr + jnp.dot(hr, w2c, preferred_element_type=jnp.float32)
                plf = plf + jnp.dot(hl, w2c, preferred_element_type=jnp.float32)
                return pr, plf
            z = jnp.zeros((MH, D), jnp.float32)
            return lax.fori_loop(0, NFC, fc_body, (z, z))

        xR_hop(0).start()
        xL_hop(0).start()

        for k in range(NH):
            xR_hop(k).wait_recv()
            xL_hop(k).wait_recv()

            if k + 1 < NH:
                if k + 1 >= 2:
                    pl.semaphore_wait(xcredR, 1)
                    pl.semaphore_wait(xcredL, 1)
                xR_hop(k + 1).start()
                xL_hop(k + 1).start()

            pr, plf = partials(xbR[k % 2], xbL[k % 2])

            if k >= 1:
                aR_hop(k - 1).wait_recv()
                aL_hop(k - 1).wait_recv()
                aR_hop(k - 1).wait_send()
                aL_hop(k - 1).wait_send()
                asR[...] = abR[(k - 1) % 2] + pr
                asL[...] = abL[(k - 1) % 2] + plf
                if k <= 5:
                    pl.semaphore_signal(acredR, inc=1, device_id=(left,),
                                        device_id_type=_MESH)
                    pl.semaphore_signal(acredL, inc=1, device_id=(right,),
                                        device_id_type=_MESH)
            else:
                asR[...] = pr
                asL[...] = plf

            if k >= 2:
                pl.semaphore_wait(acredR, 1)
                pl.semaphore_wait(acredL, 1)
            aR_hop(k).start()
            aL_hop(k).start()

            if k == 0:
                xR_hop(0).wait_send()
                xL_hop(0).wait_send()
            if k + 1 < NH:
                xR_hop(k + 1).wait_send()
                xL_hop(k + 1).wait_send()
                if k <= 4:
                    pl.semaphore_signal(xcredR, inc=1, device_id=(left,),
                                        device_id_type=_MESH)
                    pl.semaphore_signal(xcredL, inc=1, device_id=(right,),
                                        device_id_type=_MESH)

        pr_own, pl_own = partials(x_ref[pl.ds(0, MH), :],
                                  x_ref[pl.ds(MH, MH), :])

        aR_hop(NH - 1).wait_recv()
        aL_hop(NH - 1).wait_recv()
        out_ref[pl.ds(0, MH), :] = abR[(NH - 1) % 2] + pr_own
        out_ref[pl.ds(MH, MH), :] = abL[(NH - 1) % 2] + pl_own
        aR_hop(NH - 1).wait_send()
        aL_hop(NH - 1).wait_send()

    return pl.pallas_call(
        body,
        out_shape=jax.ShapeDtypeStruct((M, D), jnp.float32),
        in_specs=[
            pl.BlockSpec(memory_space=pltpu.MemorySpace.VMEM),
            pl.BlockSpec(memory_space=pl.ANY),
            pl.BlockSpec(memory_space=pl.ANY),
        ],
        out_specs=pl.BlockSpec(memory_space=pltpu.MemorySpace.VMEM),
        scratch_shapes=[
            pltpu.VMEM((2, MH, D), jnp.float32),
            pltpu.VMEM((2, MH, D), jnp.float32),
            pltpu.VMEM((2, MH, D), jnp.float32),
            pltpu.VMEM((2, MH, D), jnp.float32),
            pltpu.VMEM((MH, D), jnp.float32),
            pltpu.VMEM((MH, D), jnp.float32),
            pltpu.VMEM((2, D, FC), jnp.float32),
            pltpu.VMEM((2, FC, D), jnp.float32),
            pltpu.SemaphoreType.DMA((2,)),
            pltpu.SemaphoreType.DMA((2,)),
            pltpu.SemaphoreType.DMA((2,)),
            pltpu.SemaphoreType.DMA((2,)),
            pltpu.SemaphoreType.DMA((2,)),
            pltpu.SemaphoreType.DMA((2,)),
            pltpu.SemaphoreType.DMA((2,)),
            pltpu.SemaphoreType.DMA((2,)),
            pltpu.SemaphoreType.DMA((2, 2)),
            pltpu.SemaphoreType.REGULAR,
            pltpu.SemaphoreType.REGULAR,
            pltpu.SemaphoreType.REGULAR,
            pltpu.SemaphoreType.REGULAR,
        ],
        compiler_params=pltpu.CompilerParams(
            collective_id=0,
            vmem_limit_bytes=56 * 1024 * 1024,
        ),
    )(x, W1, W2)


# device time: 344934 ns/iter; 2.7766x vs baseline; 1.0004x over previous
import jax
import jax.numpy as jnp
from jax import lax
from jax.experimental import pallas as pl
from jax.experimental.pallas import tpu as pltpu

N_DEV = 8
NH = N_DEV - 1
M = 1024
MH = 512
D = 1024
F = 4096
FC = 512
NFC = F // FC

_MESH = pl.DeviceIdType.MESH


def kernel(x, W1, W2):
    def body(x_ref, w1_hbm, w2_hbm, out_ref,
             xbR, xbL, abR, abL, asR, asL, w1buf, w2buf,
             xsR, xrR, xsL, xrL, asRs, arR, asLs, arL, ldma,
             xcredR, xcredL, acredR, acredL):
        my = lax.axis_index("i")

        def lookup(table, idx):
            r = jnp.int32(table[0])
            for i in range(1, N_DEV):
                r = jnp.where(idx == i, jnp.int32(table[i]), r)
            return r

        right = lookup((4, 0, 6, 2, 7, 1, 5, 3), my)
        left = lookup((1, 5, 3, 7, 0, 6, 2, 4), my)

        barrier = pltpu.get_barrier_semaphore()
        pl.semaphore_signal(barrier, inc=1, device_id=(left,),
                            device_id_type=_MESH)
        pl.semaphore_signal(barrier, inc=1, device_id=(right,),
                            device_id_type=_MESH)
        pl.semaphore_wait(barrier, 2)

        def xR_hop(k):
            src = x_ref.at[pl.ds(0, MH), :] if k == 0 else xbR.at[(k - 1) % 2]
            return pltpu.make_async_remote_copy(
                src_ref=src, dst_ref=xbR.at[k % 2],
                send_sem=xsR.at[k % 2], recv_sem=xrR.at[k % 2],
                device_id=(right,), device_id_type=_MESH)

        def xL_hop(k):
            src = x_ref.at[pl.ds(MH, MH), :] if k == 0 else xbL.at[(k - 1) % 2]
            return pltpu.make_async_remote_copy(
                src_ref=src, dst_ref=xbL.at[k % 2],
                send_sem=xsL.at[k % 2], recv_sem=xrL.at[k % 2],
                device_id=(left,), device_id_type=_MESH)

        def aR_hop(k):
            return pltpu.make_async_remote_copy(
                src_ref=asR, dst_ref=abR.at[k % 2],
                send_sem=asRs.at[k % 2], recv_sem=arR.at[k % 2],
                device_id=(right,), device_id_type=_MESH)

        def aL_hop(k):
            return pltpu.make_async_remote_copy(
                src_ref=asL, dst_ref=abL.at[k % 2],
                send_sem=asLs.at[k % 2], recv_sem=arL.at[k % 2],
                device_id=(left,), device_id_type=_MESH)

        def wchunk_dma(fc, slot):
            c1 = pltpu.make_async_copy(
                w1_hbm.at[:, pl.ds(fc * FC, FC)], w1buf.at[slot],
                ldma.at[0, slot])
            c2 = pltpu.make_async_copy(
                w2_hbm.at[pl.ds(fc * FC, FC), :], w2buf.at[slot],
                ldma.at[1, slot])
            return c1, c2

        def partials(xr_val, xl_val):
            c1, c2 = wchunk_dma(0, 0)
            c1.start()
            c2.start()

            def fc_body(fc, carry):
                pr, plf = carry
                slot = lax.rem(fc, 2)
                c1, c2 = wchunk_dma(fc, slot)
                c1.wait()
                c2.wait()

                @pl.when(fc + 1 < NFC)
                def _():
                    n1, n2 = wchunk_dma(fc + 1, lax.rem(fc + 1, 2))
                    n1.start()
                    n2.start()

                w1c = w1buf[slot]
                w2c = w2buf[slot]
                hr = jnp.dot(xr_val, w1c, preferred_element_type=jnp.float32)
                hr = hr * jax.nn.sigmoid(hr)
                hl = jnp.dot(xl_val, w1c, preferred_element_type=jnp.float32)
                hl = hl * jax.nn.sigmoid(hl)
                pr = pr + jnp.dot(hr, w2c, preferred_element_type=jnp.float32)
                plf = plf + jnp.dot(hl, w2c, preferred_element_type=jnp.float32)
                return pr, plf
            z = jnp.zeros((MH, D), jnp.float32)
            return lax.fori_loop(0, NFC, fc_body, (z, z))

        xR_hop(0).start()
        xL_hop(0).start()

        for k in range(NH):
            xR_hop(k).wait_recv()
            xL_hop(k).wait_recv()

            if k + 1 < NH:
                if k + 1 >= 2:
                    pl.semaphore_wait(xcredR, 1)
                    pl.semaphore_wait(xcredL, 1)
                xR_hop(k + 1).start()
                xL_hop(k + 1).start()

            pr, plf = partials(xbR[k % 2], xbL[k % 2])

            if k >= 1:
                aR_hop(k - 1).wait_recv()
                aL_hop(k - 1).wait_recv()
                aR_hop(k - 1).wait_send()
                aL_hop(k - 1).wait_send()
                asR[...] = abR[(k - 1) % 2] + pr
                asL[...] = abL[(k - 1) % 2] + plf
                if k <= 5:
                    pl.semaphore_signal(acredR, inc=1, device_id=(left,),
                                        device_id_type=_MESH)
                    pl.semaphore_signal(acredL, inc=1, device_id=(right,),
                                        device_id_type=_MESH)
            else:
                asR[...] = pr
                asL[...] = plf

            if k >= 2:
                pl.semaphore_wait(acredR, 1)
                pl.semaphore_wait(acredL, 1)
            aR_hop(k).start()
            aL_hop(k).start()

            if k == 0:
                xR_hop(0).wait_send()
                xL_hop(0).wait_send()
            if k + 1 < NH:
                xR_hop(k + 1).wait_send()
                xL_hop(k + 1).wait_send()
                if k <= 4:
                    pl.semaphore_signal(xcredR, inc=1, device_id=(left,),
                                        device_id_type=_MESH)
                    pl.semaphore_signal(xcredL, inc=1, device_id=(right,),
                                        device_id_type=_MESH)

        pr_own, pl_own = partials(x_ref[pl.ds(0, MH), :],
                                  x_ref[pl.ds(MH, MH), :])

        aR_hop(NH - 1).wait_recv()
        aL_hop(NH - 1).wait_recv()
        out_ref[pl.ds(0, MH), :] = abR[(NH - 1) % 2] + pr_own
        out_ref[pl.ds(MH, MH), :] = abL[(NH - 1) % 2] + pl_own
        aR_hop(NH - 1).wait_send()
        aL_hop(NH - 1).wait_send()

    return pl.pallas_call(
        body,
        out_shape=jax.ShapeDtypeStruct((M, D), jnp.float32),
        in_specs=[
            pl.BlockSpec(memory_space=pltpu.MemorySpace.VMEM),
            pl.BlockSpec(memory_space=pl.ANY),
            pl.BlockSpec(memory_space=pl.ANY),
        ],
        out_specs=pl.BlockSpec(memory_space=pltpu.MemorySpace.VMEM),
        scratch_shapes=[
            pltpu.VMEM((2, MH, D), jnp.float32),
            pltpu.VMEM((2, MH, D), jnp.float32),
            pltpu.VMEM((2, MH, D), jnp.float32),
            pltpu.VMEM((2, MH, D), jnp.float32),
            pltpu.VMEM((MH, D), jnp.float32),
            pltpu.VMEM((MH, D), jnp.float32),
            pltpu.VMEM((2, D, FC), jnp.float32),
            pltpu.VMEM((2, FC, D), jnp.float32),
            pltpu.SemaphoreType.DMA((2,)),
            pltpu.SemaphoreType.DMA((2,)),
            pltpu.SemaphoreType.DMA((2,)),
            pltpu.SemaphoreType.DMA((2,)),
            pltpu.SemaphoreType.DMA((2,)),
            pltpu.SemaphoreType.DMA((2,)),
            pltpu.SemaphoreType.DMA((2,)),
            pltpu.SemaphoreType.DMA((2,)),
            pltpu.SemaphoreType.DMA((2, 2)),
            pltpu.SemaphoreType.REGULAR,
            pltpu.SemaphoreType.REGULAR,
            pltpu.SemaphoreType.REGULAR,
            pltpu.SemaphoreType.REGULAR,
        ],
        compiler_params=pltpu.CompilerParams(
            collective_id=0,
            vmem_limit_bytes=56 * 1024 * 1024,
        ),
    )(x, W1, W2)
